# Initial kernel scaffold; baseline (speedup 1.0000x reference)
import jax
import jax.numpy as jnp
from jax.experimental import pallas as pl


def kernel(x, edge_index, W1, att_src1, att_dst1, b1, W2, att_src2, att_dst2, b2):
    raise NotImplementedError("write your pallas kernel here")


# speedup vs baseline: 12.2119x; 12.2119x over previous
import functools

import jax
import jax.numpy as jnp
from jax import lax
from jax.experimental import pallas as pl
from jax.experimental.pallas import tpu as pltpu
from jax.experimental.pallas import tpu_sc as plsc

N = 10000
D_IN = 128
HID = 128
D_OUT = 128
HEADS = 4

R = 10240
FH = 64
WACC = 80
RB = 512
G = R // RB
NTILES = 32
CH = 128
NEG = -1e30


def _tc_a_body(x_ref, w_ref, as_ref, ad_ref, hf_ref, asrc_ref, adst_ref, cm_ref):
    i = pl.program_id(0)
    h = jnp.dot(x_ref[...], w_ref[...], preferred_element_type=jnp.float32)
    hr = h.reshape(RB, HEADS, HID)
    asv = jnp.sum(hr * as_ref[:HEADS][None], axis=-1)
    adv = jnp.sum(hr * ad_ref[:HEADS][None], axis=-1)
    hf_ref[...] = hr.transpose(1, 0, 2)
    pad4 = jnp.zeros((8 - HEADS, RB), jnp.float32)
    asrc_ref[...] = jnp.concatenate([asv.T, pad4], axis=0)
    adst_ref[...] = jnp.concatenate([adv.T, pad4], axis=0)
    ms = jnp.max(asv, axis=0)
    md = jnp.max(adv, axis=0)
    upd = jnp.stack([jnp.broadcast_to(ms[hh], (128,)) for hh in range(HEADS)]
                    + [jnp.broadcast_to(md[hh], (128,)) for hh in range(HEADS)])
    prev = jnp.where(i == 0, jnp.full((8, 128), NEG, jnp.float32), cm_ref[...])
    cm_ref[...] = jnp.maximum(prev, upd)


def _tc_a(xp, W1, as1p, ad1p):
    return pl.pallas_call(
        _tc_a_body,
        grid=(G,),
        in_specs=[
            pl.BlockSpec((RB, D_IN), lambda i: (i, 0)),
            pl.BlockSpec((D_IN, HEADS * HID), lambda i: (0, 0)),
            pl.BlockSpec((8, HID), lambda i: (0, 0)),
            pl.BlockSpec((8, HID), lambda i: (0, 0)),
        ],
        out_specs=[
            pl.BlockSpec((HEADS, RB, HID), lambda i: (0, i, 0)),
            pl.BlockSpec((8, RB), lambda i: (0, i)),
            pl.BlockSpec((8, RB), lambda i: (0, i)),
            pl.BlockSpec((8, 128), lambda i: (0, 0)),
        ],
        out_shape=[
            jax.ShapeDtypeStruct((HEADS, R, HID), jnp.float32),
            jax.ShapeDtypeStruct((8, R), jnp.float32),
            jax.ShapeDtypeStruct((8, R), jnp.float32),
            jax.ShapeDtypeStruct((8, 128), jnp.float32),
        ],
    )(xp, W1, as1p, ad1p)


def _tc_b_body(p_ref, w_ref, b1_ref, as_ref, ad_ref,
               hf_ref, a2_ref, cm_ref):
    i = pl.program_id(0)
    ps = p_ref[...]
    ssum = ps[:, 0] + ps[:, 1]
    cols = []
    for h in range(HEADS):
        den = ssum[2 * h, :, FH]
        feat = jnp.concatenate(
            [ssum[2 * h, :, :FH], ssum[2 * h + 1, :, :FH]], axis=-1)
        o = feat / (den[:, None] + 1e-16) + b1_ref[h][None, :]
        cols.append(jnp.maximum(o, 0.0))
    mat = jnp.concatenate(cols, axis=-1)
    h2 = jnp.dot(mat, w_ref[...], preferred_element_type=jnp.float32)
    asv = jnp.sum(h2 * as_ref[0][None], axis=-1)
    adv = jnp.sum(h2 * ad_ref[0][None], axis=-1)
    hf_ref[...] = h2
    pad6 = jnp.zeros((6, RB), jnp.float32)
    a2_ref[...] = jnp.concatenate([asv[None], adv[None], pad6], axis=0)
    neg_row = jnp.full((128,), NEG, jnp.float32)
    upd = jnp.stack([jnp.broadcast_to(jnp.max(asv), (128,))] + [neg_row] * 3
                    + [jnp.broadcast_to(jnp.max(adv), (128,))] + [neg_row] * 3)
    prev = jnp.where(i == 0, jnp.full((8, 128), NEG, jnp.float32), cm_ref[...])
    cm_ref[...] = jnp.maximum(prev, upd)


def _tc_b(p1, W2, b1p, as2p, ad2p):
    return pl.pallas_call(
        _tc_b_body,
        grid=(G,),
        in_specs=[
            pl.BlockSpec((2 * HEADS, 2, RB, WACC), lambda i: (0, 0, i, 0)),
            pl.BlockSpec((HEADS * HID, D_OUT), lambda i: (0, 0)),
            pl.BlockSpec((8, 128), lambda i: (0, 0)),
            pl.BlockSpec((8, D_OUT), lambda i: (0, 0)),
            pl.BlockSpec((8, D_OUT), lambda i: (0, 0)),
        ],
        out_specs=[
            pl.BlockSpec((RB, D_OUT), lambda i: (i, 0)),
            pl.BlockSpec((8, RB), lambda i: (0, i)),
            pl.BlockSpec((8, 128), lambda i: (0, 0)),
        ],
        out_shape=[
            jax.ShapeDtypeStruct((R, D_OUT), jnp.float32),
            jax.ShapeDtypeStruct((8, R), jnp.float32),
            jax.ShapeDtypeStruct((8, 128), jnp.float32),
        ],
    )(p1, W2, b1p, as2p, ad2p)


def _tc_c_body(p_ref, b2_ref, o_ref):
    ps = p_ref[...]
    ssum = ps[:, 0] + ps[:, 1]
    den = ssum[0, :, FH]
    feat = jnp.concatenate([ssum[0, :, :FH], ssum[1, :, :FH]], axis=-1)
    o_ref[...] = feat / (den[:, None] + 1e-16) + b2_ref[0][None]


def _tc_c(p2, b2p):
    return pl.pallas_call(
        _tc_c_body,
        grid=(G,),
        in_specs=[
            pl.BlockSpec((2, 2, RB, WACC), lambda i: (0, 0, i, 0)),
            pl.BlockSpec((8, D_OUT), lambda i: (0, 0)),
        ],
        out_specs=pl.BlockSpec((RB, D_OUT), lambda i: (i, 0)),
        out_shape=jax.ShapeDtypeStruct((R, D_OUT), jnp.float32),
    )(p2, b2p)


def _make_sc_agg(H, K):
    TR = R // 16

    mesh = plsc.VectorSubcoreMesh(core_axis_name="c", subcore_axis_name="s")

    @functools.partial(
        pl.kernel,
        out_type=jax.ShapeDtypeStruct((2 * H, 2, R, WACC), jnp.float32),
        mesh=mesh,
        compiler_params=pltpu.CompilerParams(needs_layout_passes=False,
                                             use_tc_tiling_on_sc=False),
        scratch_types=[
            pltpu.VMEM((K, CH), jnp.int32),
            pltpu.VMEM((K, CH), jnp.int32),
            pltpu.VMEM((K, CH), jnp.int32),
            pltpu.VMEM((R,), jnp.float32),
            pltpu.VMEM((R,), jnp.float32),
            pltpu.VMEM((CH, WACC), jnp.float32),
            pltpu.VMEM((CH,), jnp.float32),
            pltpu.VMEM((8, 128), jnp.float32),
            pltpu.VMEM_SHARED((R, WACC), jnp.float32),
            pltpu.SemaphoreType.DMA,
        ],
    )
    def k(haug, asrc_hbm, adst_hbm, cm_hbm, src3, dst3, srcoff4, zeros_hbm,
          outP, src_b, dst_b, soff_b, at_b, ad_b, rows, exb, cmv, acc, sem):
        c = lax.axis_index("c")
        s = lax.axis_index("s")
        w = c * 16 + s
        pltpu.sync_copy(src3.at[w], src_b)
        pltpu.sync_copy(dst3.at[w], dst_b)
        pltpu.sync_copy(cm_hbm, cmv)

        for h in range(H):
            pltpu.sync_copy(asrc_hbm.at[h], at_b)
            pltpu.sync_copy(adst_hbm.at[h], ad_b)
            msb = cmv[h, pl.ds(0, 16)]
            mdb = cmv[HEADS + h, pl.ds(0, 16)] if H > 1 else cmv[4, pl.ds(0, 16)]
            cb = msb + mdb
            cb = jnp.maximum(cb, 0.2 * cb)

            for half in range(2):
                t = 2 * h + half
                pltpu.sync_copy(srcoff4.at[t, w], soff_b)
                pltpu.sync_copy(zeros_hbm, rows)
                for z in range(TR // CH):
                    pltpu.sync_copy(rows, acc.at[pl.ds(s * TR + z * CH, CH)])
                plsc.subcore_barrier()

                def chunk(kk, carry):
                    for j8 in range(CH // 16):
                        s16 = src_b[kk, pl.ds(j8 * 16, 16)]
                        d16 = dst_b[kk, pl.ds(j8 * 16, 16)]
                        av = plsc.load_gather(at_b, [s16])
                        bv = plsc.load_gather(ad_b, [d16])
                        e = av + bv
                        e = jnp.maximum(e, 0.2 * e) - cb
                        exb[pl.ds(j8 * 16, 16)] = jnp.exp(e)
                    pltpu.async_copy(haug.at[soff_b.at[kk]], rows, sem).wait()

                    def srow(j, cr):
                        jb = jnp.full((16,), j, jnp.int32)
                        eb = plsc.load_gather(exb, [jb])
                        for cc in range(WACC // 16):
                            v = rows[j, pl.ds(cc * 16, 16)]
                            rows[j, pl.ds(cc * 16, 16)] = v * eb
                        return cr

                    lax.fori_loop(0, CH, srow, 0)
                    pltpu.sync_copy(rows, acc.at[dst_b.at[kk]], add=True)
                    return carry

                lax.fori_loop(0, K, chunk, 0)
                plsc.subcore_barrier()
                for z in range(TR // CH):
                    pltpu.sync_copy(acc.at[pl.ds(s * TR + z * CH, CH)], rows)
                    pltpu.sync_copy(rows,
                                    outP.at[t, c, pl.ds(s * TR + z * CH, CH)])
                plsc.subcore_barrier()

    return k


def kernel(x, edge_index, W1, att_src1, att_dst1, b1, W2, att_src2, att_dst2, b2):
    n = x.shape[0]
    e = edge_index.shape[1]
    e_tot = e + n
    per_tile = -(-e_tot // (NTILES * CH)) * CH
    K = per_tile // CH
    e_pad = per_tile * NTILES

    loops = jnp.arange(n, dtype=jnp.int32)
    src = jnp.concatenate([edge_index[0].astype(jnp.int32), loops,
                           jnp.full((e_pad - e_tot,), n, jnp.int32)])
    dst = jnp.concatenate([edge_index[1].astype(jnp.int32), loops,
                           jnp.full((e_pad - e_tot,), n, jnp.int32)])
    src3 = src.reshape(NTILES, K, CH)
    dst3 = dst.reshape(NTILES, K, CH)
    soff1 = src3[None] + (jnp.arange(2 * HEADS, dtype=jnp.int32) * R
                          )[:, None, None, None]
    soff2 = src3[None] + (jnp.arange(2, dtype=jnp.int32) * R
                          )[:, None, None, None]

    xp = jnp.pad(x, ((0, R - n), (0, 0)))
    as1p = jnp.pad(att_src1, ((0, 8 - HEADS), (0, 0)))
    ad1p = jnp.pad(att_dst1, ((0, 8 - HEADS), (0, 0)))
    as2p = jnp.pad(att_src2, ((0, 7), (0, 0)))
    ad2p = jnp.pad(att_dst2, ((0, 7), (0, 0)))
    b1p = jnp.pad(b1.reshape(HEADS, HID), ((0, 8 - HEADS), (0, 0)))
    b2p = jnp.pad(b2.reshape(1, D_OUT), ((0, 7), (0, 0)))

    ones_col = (jnp.arange(R) < n).astype(jnp.float32).reshape(R, 1)
    zpad15 = jnp.zeros((R, WACC - FH - 1), jnp.float32)
    zpad16 = jnp.zeros((R, WACC - FH), jnp.float32)
    zeros_tile = jnp.zeros((CH, WACC), jnp.float32)

    hf1, a_src1n, a_dst1n, cm1 = _tc_a(xp, W1, as1p, ad1p)
    tabA = jnp.concatenate(
        [hf1[:, :, :FH], jnp.broadcast_to(ones_col, (HEADS, R, 1)),
         jnp.broadcast_to(zpad15, (HEADS, R, WACC - FH - 1))], axis=2)
    tabB = jnp.concatenate(
        [hf1[:, :, FH:], jnp.broadcast_to(zpad16, (HEADS, R, WACC - FH))],
        axis=2)
    haug1 = jnp.stack([tabA, tabB], axis=1).reshape(2 * HEADS * R, WACC)
    sc1 = _make_sc_agg(HEADS, K)
    p1 = sc1(haug1, a_src1n[:HEADS], a_dst1n[:HEADS], cm1, src3, dst3,
             soff1, zeros_tile)

    hf2, a2, cm2 = _tc_b(p1, W2, b1p, as2p, ad2p)
    haug2 = jnp.stack(
        [jnp.concatenate([hf2[:, :FH], ones_col, zpad15], axis=1),
         jnp.concatenate([hf2[:, FH:], zpad16], axis=1)], axis=0
    ).reshape(2 * R, WACC)
    sc2 = _make_sc_agg(1, K)
    p2 = sc2(haug2, a2[0:1], a2[1:2], cm2, src3, dst3, soff2, zeros_tile)

    out = _tc_c(p2, b2p)
    return out[:n]


# speedup vs baseline: 12.6509x; 1.0359x over previous
import functools

import jax
import jax.numpy as jnp
from jax import lax
from jax.experimental import pallas as pl
from jax.experimental.pallas import tpu as pltpu
from jax.experimental.pallas import tpu_sc as plsc

N = 10000
D_IN = 128
HID = 128
D_OUT = 128
HEADS = 4

R = 10240
FH = 64
WACC = 80
RB = 512
G = R // RB
NTILES = 32
CH = 128
NEG = -1e30


def _tc_a_body(x_ref, w_ref, as_ref, ad_ref, hf_ref, asrc_ref, adst_ref, cm_ref):
    i = pl.program_id(0)
    h = jnp.dot(x_ref[...], w_ref[...], preferred_element_type=jnp.float32)
    hr = h.reshape(RB, HEADS, HID)
    asv = jnp.sum(hr * as_ref[:HEADS][None], axis=-1)
    adv = jnp.sum(hr * ad_ref[:HEADS][None], axis=-1)
    hf_ref[...] = hr.transpose(1, 0, 2)
    pad4 = jnp.zeros((8 - HEADS, RB), jnp.float32)
    asrc_ref[...] = jnp.concatenate([asv.T, pad4], axis=0)
    adst_ref[...] = jnp.concatenate([adv.T, pad4], axis=0)
    ms = jnp.max(asv, axis=0)
    md = jnp.max(adv, axis=0)
    upd = jnp.stack([jnp.broadcast_to(ms[hh], (128,)) for hh in range(HEADS)]
                    + [jnp.broadcast_to(md[hh], (128,)) for hh in range(HEADS)])
    prev = jnp.where(i == 0, jnp.full((8, 128), NEG, jnp.float32), cm_ref[...])
    cm_ref[...] = jnp.maximum(prev, upd)


def _tc_a(xp, W1, as1p, ad1p):
    return pl.pallas_call(
        _tc_a_body,
        grid=(G,),
        in_specs=[
            pl.BlockSpec((RB, D_IN), lambda i: (i, 0)),
            pl.BlockSpec((D_IN, HEADS * HID), lambda i: (0, 0)),
            pl.BlockSpec((8, HID), lambda i: (0, 0)),
            pl.BlockSpec((8, HID), lambda i: (0, 0)),
        ],
        out_specs=[
            pl.BlockSpec((HEADS, RB, HID), lambda i: (0, i, 0)),
            pl.BlockSpec((8, RB), lambda i: (0, i)),
            pl.BlockSpec((8, RB), lambda i: (0, i)),
            pl.BlockSpec((8, 128), lambda i: (0, 0)),
        ],
        out_shape=[
            jax.ShapeDtypeStruct((HEADS, R, HID), jnp.float32),
            jax.ShapeDtypeStruct((8, R), jnp.float32),
            jax.ShapeDtypeStruct((8, R), jnp.float32),
            jax.ShapeDtypeStruct((8, 128), jnp.float32),
        ],
    )(xp, W1, as1p, ad1p)


def _tc_b_body(p_ref, w_ref, b1_ref, as_ref, ad_ref,
               hf_ref, a2_ref, cm_ref):
    i = pl.program_id(0)
    ps = p_ref[...]
    ssum = ps[:, 0] + ps[:, 1]
    cols = []
    for h in range(HEADS):
        den = ssum[2 * h, :, FH]
        feat = jnp.concatenate(
            [ssum[2 * h, :, :FH], ssum[2 * h + 1, :, :FH]], axis=-1)
        o = feat / (den[:, None] + 1e-16) + b1_ref[h][None, :]
        cols.append(jnp.maximum(o, 0.0))
    mat = jnp.concatenate(cols, axis=-1)
    h2 = jnp.dot(mat, w_ref[...], preferred_element_type=jnp.float32)
    asv = jnp.sum(h2 * as_ref[0][None], axis=-1)
    adv = jnp.sum(h2 * ad_ref[0][None], axis=-1)
    hf_ref[...] = h2
    pad6 = jnp.zeros((6, RB), jnp.float32)
    a2_ref[...] = jnp.concatenate([asv[None], adv[None], pad6], axis=0)
    neg_row = jnp.full((128,), NEG, jnp.float32)
    upd = jnp.stack([jnp.broadcast_to(jnp.max(asv), (128,))] + [neg_row] * 3
                    + [jnp.broadcast_to(jnp.max(adv), (128,))] + [neg_row] * 3)
    prev = jnp.where(i == 0, jnp.full((8, 128), NEG, jnp.float32), cm_ref[...])
    cm_ref[...] = jnp.maximum(prev, upd)


def _tc_b(p1, W2, b1p, as2p, ad2p):
    return pl.pallas_call(
        _tc_b_body,
        grid=(G,),
        in_specs=[
            pl.BlockSpec((2 * HEADS, 2, RB, WACC), lambda i: (0, 0, i, 0)),
            pl.BlockSpec((HEADS * HID, D_OUT), lambda i: (0, 0)),
            pl.BlockSpec((8, 128), lambda i: (0, 0)),
            pl.BlockSpec((8, D_OUT), lambda i: (0, 0)),
            pl.BlockSpec((8, D_OUT), lambda i: (0, 0)),
        ],
        out_specs=[
            pl.BlockSpec((RB, D_OUT), lambda i: (i, 0)),
            pl.BlockSpec((8, RB), lambda i: (0, i)),
            pl.BlockSpec((8, 128), lambda i: (0, 0)),
        ],
        out_shape=[
            jax.ShapeDtypeStruct((R, D_OUT), jnp.float32),
            jax.ShapeDtypeStruct((8, R), jnp.float32),
            jax.ShapeDtypeStruct((8, 128), jnp.float32),
        ],
    )(p1, W2, b1p, as2p, ad2p)


def _tc_c_body(p_ref, b2_ref, o_ref):
    ps = p_ref[...]
    ssum = ps[:, 0] + ps[:, 1]
    den = ssum[0, :, FH]
    feat = jnp.concatenate([ssum[0, :, :FH], ssum[1, :, :FH]], axis=-1)
    o_ref[...] = feat / (den[:, None] + 1e-16) + b2_ref[0][None]


def _tc_c(p2, b2p):
    return pl.pallas_call(
        _tc_c_body,
        grid=(G,),
        in_specs=[
            pl.BlockSpec((2, 2, RB, WACC), lambda i: (0, 0, i, 0)),
            pl.BlockSpec((8, D_OUT), lambda i: (0, 0)),
        ],
        out_specs=pl.BlockSpec((RB, D_OUT), lambda i: (i, 0)),
        out_shape=jax.ShapeDtypeStruct((R, D_OUT), jnp.float32),
    )(p2, b2p)


def _make_sc_agg(H, K):
    TR = R // 16

    mesh = plsc.VectorSubcoreMesh(core_axis_name="c", subcore_axis_name="s")

    @functools.partial(
        pl.kernel,
        out_type=jax.ShapeDtypeStruct((2 * H, 2, R, WACC), jnp.float32),
        mesh=mesh,
        compiler_params=pltpu.CompilerParams(needs_layout_passes=False,
                                             use_tc_tiling_on_sc=False),
        scratch_types=[
            pltpu.VMEM((K, CH), jnp.int32),
            pltpu.VMEM((K, CH), jnp.int32),
            pltpu.VMEM((K, CH), jnp.int32),
            pltpu.VMEM((R,), jnp.float32),
            pltpu.VMEM((R,), jnp.float32),
            pltpu.VMEM((CH, WACC), jnp.float32),
            pltpu.VMEM((CH,), jnp.float32),
            pltpu.VMEM((8, 128), jnp.float32),
            pltpu.VMEM_SHARED((R, WACC), jnp.float32),
            pltpu.SemaphoreType.DMA,
        ],
    )
    def k(haug, asrc_hbm, adst_hbm, cm_hbm, src3, dst3, srcoff4, zeros_hbm,
          outP, src_b, dst_b, soff_b, at_b, ad_b, rows, exb, cmv, acc, sem):
        c = lax.axis_index("c")
        s = lax.axis_index("s")
        w = c * 16 + s
        pltpu.sync_copy(src3.at[w], src_b)
        pltpu.sync_copy(dst3.at[w], dst_b)
        pltpu.sync_copy(cm_hbm, cmv)

        for h in range(H):
            pltpu.sync_copy(asrc_hbm.at[h], at_b)
            pltpu.sync_copy(adst_hbm.at[h], ad_b)
            msb = cmv[h, pl.ds(0, 16)]
            mdb = cmv[HEADS + h, pl.ds(0, 16)] if H > 1 else cmv[4, pl.ds(0, 16)]
            cb = msb + mdb
            cb = jnp.maximum(cb, 0.2 * cb)

            for half in range(2):
                t = 2 * h + half
                pltpu.sync_copy(srcoff4.at[t, w], soff_b)
                pltpu.sync_copy(zeros_hbm, rows)
                for z in range(TR // CH):
                    pltpu.sync_copy(rows, acc.at[pl.ds(s * TR + z * CH, CH)])
                plsc.subcore_barrier()

                def chunk(kk, carry):
                    cp = pltpu.async_copy(haug.at[soff_b.at[kk]], rows, sem)
                    for j8 in range(CH // 16):
                        s16 = src_b[kk, pl.ds(j8 * 16, 16)]
                        d16 = dst_b[kk, pl.ds(j8 * 16, 16)]
                        av = plsc.load_gather(at_b, [s16])
                        bv = plsc.load_gather(ad_b, [d16])
                        e = av + bv
                        e = jnp.maximum(e, 0.2 * e) - cb
                        exb[pl.ds(j8 * 16, 16)] = jnp.exp(e)
                    cp.wait()

                    def srow(q, cr):
                        for u in range(4):
                            j = q * 4 + u
                            jb = jnp.full((16,), j, jnp.int32)
                            eb = plsc.load_gather(exb, [jb])
                            for cc in range(WACC // 16):
                                v = rows[j, pl.ds(cc * 16, 16)]
                                rows[j, pl.ds(cc * 16, 16)] = v * eb
                        return cr

                    lax.fori_loop(0, CH // 4, srow, 0)
                    pltpu.sync_copy(rows, acc.at[dst_b.at[kk]], add=True)
                    return carry

                lax.fori_loop(0, K, chunk, 0)
                plsc.subcore_barrier()
                for z in range(TR // CH):
                    pltpu.sync_copy(acc.at[pl.ds(s * TR + z * CH, CH)], rows)
                    pltpu.sync_copy(rows,
                                    outP.at[t, c, pl.ds(s * TR + z * CH, CH)])
                plsc.subcore_barrier()

    return k


def kernel(x, edge_index, W1, att_src1, att_dst1, b1, W2, att_src2, att_dst2, b2):
    n = x.shape[0]
    e = edge_index.shape[1]
    e_tot = e + n
    per_tile = -(-e_tot // (NTILES * CH)) * CH
    K = per_tile // CH
    e_pad = per_tile * NTILES

    loops = jnp.arange(n, dtype=jnp.int32)
    src = jnp.concatenate([edge_index[0].astype(jnp.int32), loops,
                           jnp.full((e_pad - e_tot,), n, jnp.int32)])
    dst = jnp.concatenate([edge_index[1].astype(jnp.int32), loops,
                           jnp.full((e_pad - e_tot,), n, jnp.int32)])
    src3 = src.reshape(NTILES, K, CH)
    dst3 = dst.reshape(NTILES, K, CH)
    soff1 = src3[None] + (jnp.arange(2 * HEADS, dtype=jnp.int32) * R
                          )[:, None, None, None]
    soff2 = src3[None] + (jnp.arange(2, dtype=jnp.int32) * R
                          )[:, None, None, None]

    xp = jnp.pad(x, ((0, R - n), (0, 0)))
    as1p = jnp.pad(att_src1, ((0, 8 - HEADS), (0, 0)))
    ad1p = jnp.pad(att_dst1, ((0, 8 - HEADS), (0, 0)))
    as2p = jnp.pad(att_src2, ((0, 7), (0, 0)))
    ad2p = jnp.pad(att_dst2, ((0, 7), (0, 0)))
    b1p = jnp.pad(b1.reshape(HEADS, HID), ((0, 8 - HEADS), (0, 0)))
    b2p = jnp.pad(b2.reshape(1, D_OUT), ((0, 7), (0, 0)))

    ones_col = (jnp.arange(R) < n).astype(jnp.float32).reshape(R, 1)
    zpad15 = jnp.zeros((R, WACC - FH - 1), jnp.float32)
    zpad16 = jnp.zeros((R, WACC - FH), jnp.float32)
    zeros_tile = jnp.zeros((CH, WACC), jnp.float32)

    hf1, a_src1n, a_dst1n, cm1 = _tc_a(xp, W1, as1p, ad1p)
    tabA = jnp.concatenate(
        [hf1[:, :, :FH], jnp.broadcast_to(ones_col, (HEADS, R, 1)),
         jnp.broadcast_to(zpad15, (HEADS, R, WACC - FH - 1))], axis=2)
    tabB = jnp.concatenate(
        [hf1[:, :, FH:], jnp.broadcast_to(zpad16, (HEADS, R, WACC - FH))],
        axis=2)
    haug1 = jnp.stack([tabA, tabB], axis=1).reshape(2 * HEADS * R, WACC)
    sc1 = _make_sc_agg(HEADS, K)
    p1 = sc1(haug1, a_src1n[:HEADS], a_dst1n[:HEADS], cm1, src3, dst3,
             soff1, zeros_tile)

    hf2, a2, cm2 = _tc_b(p1, W2, b1p, as2p, ad2p)
    haug2 = jnp.stack(
        [jnp.concatenate([hf2[:, :FH], ones_col, zpad15], axis=1),
         jnp.concatenate([hf2[:, FH:], zpad16], axis=1)], axis=0
    ).reshape(2 * R, WACC)
    sc2 = _make_sc_agg(1, K)
    p2 = sc2(haug2, a2[0:1], a2[1:2], cm2, src3, dst3, soff2, zeros_tile)

    out = _tc_c(p2, b2p)
    return out[:n]


# speedup vs baseline: 14.4104x; 1.1391x over previous
import functools

import jax
import jax.numpy as jnp
from jax import lax
from jax.experimental import pallas as pl
from jax.experimental.pallas import tpu as pltpu
from jax.experimental.pallas import tpu_sc as plsc

N = 10000
D_IN = 128
HID = 128
D_OUT = 128
HEADS = 4

R = 10240
FH = 64
WACC = 80
RB = 512
G = R // RB
NTILES = 32
CH = 128
NEG = -1e30


def _tc_a_body(x_ref, w_ref, as_ref, ad_ref, hf_ref, asrc_ref, adst_ref, cm_ref):
    i = pl.program_id(0)
    h = jnp.dot(x_ref[...], w_ref[...], preferred_element_type=jnp.float32)
    hr = h.reshape(RB, HEADS, HID)
    asv = jnp.sum(hr * as_ref[:HEADS][None], axis=-1)
    adv = jnp.sum(hr * ad_ref[:HEADS][None], axis=-1)
    hf_ref[...] = hr.transpose(1, 0, 2)
    pad4 = jnp.zeros((8 - HEADS, RB), jnp.float32)
    asrc_ref[...] = jnp.concatenate([asv.T, pad4], axis=0)
    adst_ref[...] = jnp.concatenate([adv.T, pad4], axis=0)
    ms = jnp.max(asv, axis=0)
    md = jnp.max(adv, axis=0)
    upd = jnp.stack([jnp.broadcast_to(ms[hh], (128,)) for hh in range(HEADS)]
                    + [jnp.broadcast_to(md[hh], (128,)) for hh in range(HEADS)])
    prev = jnp.where(i == 0, jnp.full((8, 128), NEG, jnp.float32), cm_ref[...])
    cm_ref[...] = jnp.maximum(prev, upd)


def _tc_a(xp, W1, as1p, ad1p):
    return pl.pallas_call(
        _tc_a_body,
        grid=(G,),
        in_specs=[
            pl.BlockSpec((RB, D_IN), lambda i: (i, 0)),
            pl.BlockSpec((D_IN, HEADS * HID), lambda i: (0, 0)),
            pl.BlockSpec((8, HID), lambda i: (0, 0)),
            pl.BlockSpec((8, HID), lambda i: (0, 0)),
        ],
        out_specs=[
            pl.BlockSpec((HEADS, RB, HID), lambda i: (0, i, 0)),
            pl.BlockSpec((8, RB), lambda i: (0, i)),
            pl.BlockSpec((8, RB), lambda i: (0, i)),
            pl.BlockSpec((8, 128), lambda i: (0, 0)),
        ],
        out_shape=[
            jax.ShapeDtypeStruct((HEADS, R, HID), jnp.float32),
            jax.ShapeDtypeStruct((8, R), jnp.float32),
            jax.ShapeDtypeStruct((8, R), jnp.float32),
            jax.ShapeDtypeStruct((8, 128), jnp.float32),
        ],
    )(xp, W1, as1p, ad1p)


def _tc_b_body(p_ref, w_ref, b1_ref, as_ref, ad_ref,
               hf_ref, a2_ref, cm_ref):
    i = pl.program_id(0)
    ps = p_ref[...]
    ssum = ps[:, 0] + ps[:, 1]
    cols = []
    for h in range(HEADS):
        den = ssum[2 * h, :, FH]
        feat = jnp.concatenate(
            [ssum[2 * h, :, :FH], ssum[2 * h + 1, :, :FH]], axis=-1)
        o = feat / (den[:, None] + 1e-16) + b1_ref[h][None, :]
        cols.append(jnp.maximum(o, 0.0))
    mat = jnp.concatenate(cols, axis=-1)
    h2 = jnp.dot(mat, w_ref[...], preferred_element_type=jnp.float32)
    asv = jnp.sum(h2 * as_ref[0][None], axis=-1)
    adv = jnp.sum(h2 * ad_ref[0][None], axis=-1)
    hf_ref[...] = h2
    pad6 = jnp.zeros((6, RB), jnp.float32)
    a2_ref[...] = jnp.concatenate([asv[None], adv[None], pad6], axis=0)
    neg_row = jnp.full((128,), NEG, jnp.float32)
    upd = jnp.stack([jnp.broadcast_to(jnp.max(asv), (128,))] + [neg_row] * 3
                    + [jnp.broadcast_to(jnp.max(adv), (128,))] + [neg_row] * 3)
    prev = jnp.where(i == 0, jnp.full((8, 128), NEG, jnp.float32), cm_ref[...])
    cm_ref[...] = jnp.maximum(prev, upd)


def _tc_b(p1, W2, b1p, as2p, ad2p):
    return pl.pallas_call(
        _tc_b_body,
        grid=(G,),
        in_specs=[
            pl.BlockSpec((2 * HEADS, 2, RB, WACC), lambda i: (0, 0, i, 0)),
            pl.BlockSpec((HEADS * HID, D_OUT), lambda i: (0, 0)),
            pl.BlockSpec((8, 128), lambda i: (0, 0)),
            pl.BlockSpec((8, D_OUT), lambda i: (0, 0)),
            pl.BlockSpec((8, D_OUT), lambda i: (0, 0)),
        ],
        out_specs=[
            pl.BlockSpec((RB, D_OUT), lambda i: (i, 0)),
            pl.BlockSpec((8, RB), lambda i: (0, i)),
            pl.BlockSpec((8, 128), lambda i: (0, 0)),
        ],
        out_shape=[
            jax.ShapeDtypeStruct((R, D_OUT), jnp.float32),
            jax.ShapeDtypeStruct((8, R), jnp.float32),
            jax.ShapeDtypeStruct((8, 128), jnp.float32),
        ],
    )(p1, W2, b1p, as2p, ad2p)


def _tc_c_body(p_ref, b2_ref, o_ref):
    ps = p_ref[...]
    ssum = ps[:, 0] + ps[:, 1]
    den = ssum[0, :, FH]
    feat = jnp.concatenate([ssum[0, :, :FH], ssum[1, :, :FH]], axis=-1)
    o_ref[...] = feat / (den[:, None] + 1e-16) + b2_ref[0][None]


def _tc_c(p2, b2p):
    return pl.pallas_call(
        _tc_c_body,
        grid=(G,),
        in_specs=[
            pl.BlockSpec((2, 2, RB, WACC), lambda i: (0, 0, i, 0)),
            pl.BlockSpec((8, D_OUT), lambda i: (0, 0)),
        ],
        out_specs=pl.BlockSpec((RB, D_OUT), lambda i: (i, 0)),
        out_shape=jax.ShapeDtypeStruct((R, D_OUT), jnp.float32),
    )(p2, b2p)


def _make_sc_agg(H, K):
    TR = R // 16

    mesh = plsc.VectorSubcoreMesh(core_axis_name="c", subcore_axis_name="s")

    @functools.partial(
        pl.kernel,
        out_type=jax.ShapeDtypeStruct((2 * H, 2, R, WACC), jnp.float32),
        mesh=mesh,
        compiler_params=pltpu.CompilerParams(needs_layout_passes=False,
                                             use_tc_tiling_on_sc=False),
        scratch_types=[
            pltpu.VMEM((K, CH), jnp.int32),
            pltpu.VMEM((K, CH), jnp.int32),
            pltpu.VMEM((K, CH), jnp.int32),
            pltpu.VMEM((R,), jnp.float32),
            pltpu.VMEM((R,), jnp.float32),
            pltpu.VMEM((CH, WACC), jnp.float32),
            pltpu.VMEM((CH, WACC), jnp.float32),
            pltpu.VMEM((CH,), jnp.float32),
            pltpu.VMEM((8, 128), jnp.float32),
            pltpu.VMEM_SHARED((R, WACC), jnp.float32),
            pltpu.SemaphoreType.DMA,
        ],
    )
    def k(haug, asrc_hbm, adst_hbm, cm_hbm, src3, dst3, srcoff4, zeros_hbm,
          outP, src_b, dst_b, soff_b, at_b, ad_b, rows, rows2, exb, cmv, acc,
          sem):
        c = lax.axis_index("c")
        s = lax.axis_index("s")
        w = c * 16 + s
        pltpu.sync_copy(src3.at[w], src_b)
        pltpu.sync_copy(dst3.at[w], dst_b)
        pltpu.sync_copy(cm_hbm, cmv)

        for h in range(H):
            pltpu.sync_copy(asrc_hbm.at[h], at_b)
            pltpu.sync_copy(adst_hbm.at[h], ad_b)
            msb = cmv[h, pl.ds(0, 16)]
            mdb = cmv[HEADS + h, pl.ds(0, 16)] if H > 1 else cmv[4, pl.ds(0, 16)]
            cb = msb + mdb
            cb = jnp.maximum(cb, 0.2 * cb)

            for half in range(2):
                t = 2 * h + half
                pltpu.sync_copy(srcoff4.at[t, w], soff_b)
                pltpu.sync_copy(zeros_hbm, rows)
                for z in range(TR // CH):
                    pltpu.sync_copy(rows, acc.at[pl.ds(s * TR + z * CH, CH)])
                plsc.subcore_barrier()

                def ex_stage(kk):
                    for j8 in range(CH // 16):
                        s16 = src_b[kk, pl.ds(j8 * 16, 16)]
                        d16 = dst_b[kk, pl.ds(j8 * 16, 16)]
                        e = (plsc.load_gather(at_b, [s16])
                             + plsc.load_gather(ad_b, [d16]))
                        e = jnp.maximum(e, 0.2 * e) - cb
                        exb[pl.ds(j8 * 16, 16)] = jnp.exp(e)

                def scale_scatter(kk, rbuf):
                    def srow(q, cr):
                        for u in range(4):
                            j = q * 4 + u
                            jb = jnp.full((16,), j, jnp.int32)
                            eb = plsc.load_gather(exb, [jb])
                            for cc in range(WACC // 16):
                                v = rbuf[j, pl.ds(cc * 16, 16)]
                                rbuf[j, pl.ds(cc * 16, 16)] = v * eb
                        return cr

                    lax.fori_loop(0, CH // 4, srow, 0)
                    pltpu.sync_copy(rbuf, acc.at[dst_b.at[kk]], add=True)

                def pair(q, carry):
                    k0 = 2 * q
                    k1 = 2 * q + 1
                    cpa = pltpu.async_copy(haug.at[soff_b.at[k0]], rows, sem)
                    ex_stage(k0)
                    cpa.wait()
                    cpb = pltpu.async_copy(haug.at[soff_b.at[k1]], rows2, sem)
                    scale_scatter(k0, rows)
                    ex_stage(k1)
                    cpb.wait()
                    scale_scatter(k1, rows2)
                    return carry

                lax.fori_loop(0, K // 2, pair, 0)
                if K % 2:
                    kt = K - 1
                    cpt = pltpu.async_copy(haug.at[soff_b.at[kt]], rows, sem)
                    ex_stage(kt)
                    cpt.wait()
                    scale_scatter(kt, rows)
                plsc.subcore_barrier()
                for z in range(TR // CH):
                    pltpu.sync_copy(acc.at[pl.ds(s * TR + z * CH, CH)], rows)
                    pltpu.sync_copy(rows,
                                    outP.at[t, c, pl.ds(s * TR + z * CH, CH)])
                plsc.subcore_barrier()

    return k


def kernel(x, edge_index, W1, att_src1, att_dst1, b1, W2, att_src2, att_dst2, b2):
    n = x.shape[0]
    e = edge_index.shape[1]
    e_tot = e + n
    per_tile = -(-e_tot // (NTILES * CH)) * CH
    K = per_tile // CH
    e_pad = per_tile * NTILES

    loops = jnp.arange(n, dtype=jnp.int32)
    src = jnp.concatenate([edge_index[0].astype(jnp.int32), loops,
                           jnp.full((e_pad - e_tot,), n, jnp.int32)])
    dst = jnp.concatenate([edge_index[1].astype(jnp.int32), loops,
                           jnp.full((e_pad - e_tot,), n, jnp.int32)])
    src3 = src.reshape(NTILES, K, CH)
    dst3 = dst.reshape(NTILES, K, CH)
    soff1 = src3[None] + (jnp.arange(2 * HEADS, dtype=jnp.int32) * R
                          )[:, None, None, None]
    soff2 = src3[None] + (jnp.arange(2, dtype=jnp.int32) * R
                          )[:, None, None, None]

    xp = jnp.pad(x, ((0, R - n), (0, 0)))
    as1p = jnp.pad(att_src1, ((0, 8 - HEADS), (0, 0)))
    ad1p = jnp.pad(att_dst1, ((0, 8 - HEADS), (0, 0)))
    as2p = jnp.pad(att_src2, ((0, 7), (0, 0)))
    ad2p = jnp.pad(att_dst2, ((0, 7), (0, 0)))
    b1p = jnp.pad(b1.reshape(HEADS, HID), ((0, 8 - HEADS), (0, 0)))
    b2p = jnp.pad(b2.reshape(1, D_OUT), ((0, 7), (0, 0)))

    ones_col = (jnp.arange(R) < n).astype(jnp.float32).reshape(R, 1)
    zpad15 = jnp.zeros((R, WACC - FH - 1), jnp.float32)
    zpad16 = jnp.zeros((R, WACC - FH), jnp.float32)
    zeros_tile = jnp.zeros((CH, WACC), jnp.float32)

    hf1, a_src1n, a_dst1n, cm1 = _tc_a(xp, W1, as1p, ad1p)
    tabA = jnp.concatenate(
        [hf1[:, :, :FH], jnp.broadcast_to(ones_col, (HEADS, R, 1)),
         jnp.broadcast_to(zpad15, (HEADS, R, WACC - FH - 1))], axis=2)
    tabB = jnp.concatenate(
        [hf1[:, :, FH:], jnp.broadcast_to(zpad16, (HEADS, R, WACC - FH))],
        axis=2)
    haug1 = jnp.stack([tabA, tabB], axis=1).reshape(2 * HEADS * R, WACC)
    sc1 = _make_sc_agg(HEADS, K)
    p1 = sc1(haug1, a_src1n[:HEADS], a_dst1n[:HEADS], cm1, src3, dst3,
             soff1, zeros_tile)

    hf2, a2, cm2 = _tc_b(p1, W2, b1p, as2p, ad2p)
    haug2 = jnp.stack(
        [jnp.concatenate([hf2[:, :FH], ones_col, zpad15], axis=1),
         jnp.concatenate([hf2[:, FH:], zpad16], axis=1)], axis=0
    ).reshape(2 * R, WACC)
    sc2 = _make_sc_agg(1, K)
    p2 = sc2(haug2, a2[0:1], a2[1:2], cm2, src3, dst3, soff2, zeros_tile)

    out = _tc_c(p2, b2p)
    return out[:n]
